# Initial kernel scaffold; baseline (speedup 1.0000x reference)
import jax
import jax.numpy as jnp
from jax.experimental import pallas as pl


def kernel(x, edge_index, w1_1, b1_1, w2_1, w3_1, b3_1, w1_2, b1_2, w2_2, w3_2, b3_2):
    raise NotImplementedError("write your pallas kernel here")


# speedup vs baseline: 5.5605x; 5.5605x over previous
import functools

import jax
import jax.numpy as jnp
from jax import lax
from jax.experimental import pallas as pl
from jax.experimental.pallas import tpu as pltpu
from jax.experimental.pallas import tpu_sc as plsc

_CHUNK = 128
_NC = 2
_NS = 16
_ROWS = 1000


def _sc_scatter(n_nodes, d, n_pass, n_chunks):
    nw = _NC * _NS
    jobs = (n_chunks + nw - 1) // nw
    zr = 400
    rows_per_tile = 3200
    n_dma = rows_per_tile // zr
    mesh = plsc.VectorSubcoreMesh(core_axis_name="c", subcore_axis_name="s")

    @functools.partial(
        pl.kernel,
        out_type=jax.ShapeDtypeStruct((n_pass, _NC, n_nodes, d), jnp.float32),
        mesh=mesh,
        scratch_types=[
            pltpu.VMEM((_CHUNK,), jnp.int32),
            pltpu.VMEM((_CHUNK,), jnp.int32),
            pltpu.VMEM((_CHUNK, d), jnp.float32),
            pltpu.VMEM((zr, d), jnp.float32),
            pltpu.VMEM_SHARED((n_nodes, d), jnp.float32),
            pltpu.SemaphoreType.DMA,
        ],
        compiler_params=pltpu.CompilerParams(use_tc_tiling_on_sc=False),
    )
    def k(*refs):
        tables = refs[:n_pass]
        src_h, dst_h, z_h, out_h = refs[n_pass:n_pass + 4]
        src_v, dst_v, rows_v, buf_v, acc, sem = refs[n_pass + 4:]
        c = lax.axis_index("c")
        s = lax.axis_index("s")
        w = s * _NC + c
        row0 = s * rows_per_tile

        for p in range(n_pass):
            pltpu.sync_copy(z_h, buf_v)
            for r in range(n_dma):
                @pl.when(row0 + r * zr < n_nodes)
                def _():
                    pltpu.sync_copy(buf_v, acc.at[pl.ds(row0 + r * zr, zr)])
            plsc.subcore_barrier()

            def body(j, _):
                cid = w + nw * j

                @pl.when(cid < n_chunks)
                def _():
                    pltpu.sync_copy(src_h.at[cid], src_v)
                    pltpu.sync_copy(dst_h.at[cid], dst_v)
                    pltpu.async_copy(tables[p].at[src_v], rows_v, sem).wait()
                    pltpu.sync_copy(rows_v, acc.at[dst_v], add=True)

                return 0

            lax.fori_loop(0, jobs, body, 0)
            plsc.subcore_barrier()
            for r in range(n_dma):
                @pl.when(row0 + r * zr < n_nodes)
                def _():
                    sl = pl.ds(row0 + r * zr, zr)
                    pltpu.sync_copy(acc.at[sl], buf_v)
                    pltpu.sync_copy(buf_v, out_h.at[p, c, sl])

    return k


def _dense1(x_ref, w_ref, b_ref, oa_ref, ob_ref, oc_ref, obc_ref):
    y = jnp.dot(x_ref[...], w_ref[...], preferred_element_type=jnp.float32)
    y = y + b_ref[...]
    oa_ref[...] = y[:, 0:32]
    ob_ref[...] = y[:, 32:64]
    oc_ref[...] = y[:, 64:96]
    obc_ref[...] = y[:, 96:240]


def _mid(s_ref, bc_ref, w_ref, b_ref, t2_ref, p2_ref):
    s = s_ref[...]
    s0 = s[0, 0] + s[0, 1]
    s1 = s[1, 0] + s[1, 1]
    s2 = s[2, 0] + s[2, 1]
    deg = s0[:, 24:25]
    agg = jnp.concatenate([s0[:, 0:24], s1[:, 0:24], s2[:, 0:24]], axis=1)
    bc = bc_ref[...]
    h = bc[:, 72:144] - deg * bc[:, 0:72] + agg
    h = jnp.where(h > 0, h, jnp.exp(h) - 1.0)
    y = jnp.dot(h, w_ref[...], preferred_element_type=jnp.float32)
    y = y + b_ref[...]
    t2_ref[...] = y[:, 0:32]
    p2_ref[...] = y[:, 51:70] - deg * y[:, 32:51]


def _out(s_ref, p_ref, o_ref):
    s = s_ref[...]
    logits = p_ref[...] + (s[0, 0] + s[0, 1])[:, 0:19]
    m = jnp.max(logits, axis=1, keepdims=True)
    z = logits - m
    o_ref[...] = z - jnp.log(jnp.sum(jnp.exp(z), axis=1, keepdims=True))


def kernel(x, edge_index, w1_1, b1_1, w2_1, w3_1, b3_1,
           w1_2, b1_2, w2_2, w3_2, b3_2):
    n, f = x.shape
    e = edge_index.shape[1]
    n_chunks = e // _CHUNK
    grid = (n // _ROWS,)

    src = edge_index[0].astype(jnp.int32).reshape(n_chunks, _CHUNK)
    dst = edge_index[1].astype(jnp.int32).reshape(n_chunks, _CHUNK)
    zf = lambda *sh: jnp.zeros(sh, jnp.float32)

    w_a = jnp.concatenate(
        [w1_1[:, 0:24], zf(f, 8), w1_1[:, 24:48], zf(f, 8),
         w1_1[:, 48:72], zf(f, 8), w2_1, w3_1], axis=1)
    one = jnp.ones((1,), jnp.float32)
    b_a = jnp.concatenate(
        [b1_1[0:24], one, zf(7), b1_1[24:48], zf(8),
         b1_1[48:72], zf(8), zf(72), b3_1])[None]

    w_b = jnp.concatenate([w1_2, zf(72, 13), w2_2, w3_2], axis=1)
    b_b = jnp.concatenate([b1_2, zf(13 + 19), b3_2])[None]

    ta, tb, tc, bc = pl.pallas_call(
        _dense1,
        grid=grid,
        in_specs=[
            pl.BlockSpec((_ROWS, f), lambda i: (i, 0)),
            pl.BlockSpec((f, 240), lambda i: (0, 0)),
            pl.BlockSpec((1, 240), lambda i: (0, 0)),
        ],
        out_specs=[
            pl.BlockSpec((_ROWS, 32), lambda i: (i, 0)),
            pl.BlockSpec((_ROWS, 32), lambda i: (i, 0)),
            pl.BlockSpec((_ROWS, 32), lambda i: (i, 0)),
            pl.BlockSpec((_ROWS, 144), lambda i: (i, 0)),
        ],
        out_shape=[
            jax.ShapeDtypeStruct((n, 32), jnp.float32),
            jax.ShapeDtypeStruct((n, 32), jnp.float32),
            jax.ShapeDtypeStruct((n, 32), jnp.float32),
            jax.ShapeDtypeStruct((n, 144), jnp.float32),
        ],
    )(x, w_a, b_a)

    s1 = _sc_scatter(n, 32, 3, n_chunks)(ta, tb, tc, src, dst, zf(400, 32))

    t2, p2 = pl.pallas_call(
        _mid,
        grid=grid,
        in_specs=[
            pl.BlockSpec((3, 2, _ROWS, 32), lambda i: (0, 0, i, 0)),
            pl.BlockSpec((_ROWS, 144), lambda i: (i, 0)),
            pl.BlockSpec((72, 70), lambda i: (0, 0)),
            pl.BlockSpec((1, 70), lambda i: (0, 0)),
        ],
        out_specs=[
            pl.BlockSpec((_ROWS, 32), lambda i: (i, 0)),
            pl.BlockSpec((_ROWS, 19), lambda i: (i, 0)),
        ],
        out_shape=[
            jax.ShapeDtypeStruct((n, 32), jnp.float32),
            jax.ShapeDtypeStruct((n, 19), jnp.float32),
        ],
    )(s1, bc, w_b, b_b)

    s2 = _sc_scatter(n, 32, 1, n_chunks)(t2, src, dst, zf(400, 32))

    return pl.pallas_call(
        _out,
        grid=grid,
        in_specs=[
            pl.BlockSpec((1, 2, _ROWS, 32), lambda i: (0, 0, i, 0)),
            pl.BlockSpec((_ROWS, 19), lambda i: (i, 0)),
        ],
        out_specs=pl.BlockSpec((_ROWS, 19), lambda i: (i, 0)),
        out_shape=jax.ShapeDtypeStruct((n, 19), jnp.float32),
    )(s2, p2)


# speedup vs baseline: 11.8572x; 2.1324x over previous
import functools

import jax
import jax.numpy as jnp
from jax import lax
from jax.experimental import pallas as pl
from jax.experimental.pallas import tpu as pltpu
from jax.experimental.pallas import tpu_sc as plsc

_CHUNK = 128
_NC = 2
_NS = 16
_ROWS = 1000


def _sc_scatter(n_nodes, d, n_pass, jobs):
    nw = _NC * _NS
    nbuf = 4
    sc = 28
    n_super = jobs // sc
    assert jobs == n_super * sc
    rows_per_tile = 3200
    acc_rows = 50048
    mesh = plsc.VectorSubcoreMesh(core_axis_name="c", subcore_axis_name="s")

    @functools.partial(
        pl.kernel,
        out_type=jax.ShapeDtypeStruct((n_pass, _NC, n_nodes, d), jnp.float32),
        mesh=mesh,
        scratch_types=[
            pltpu.VMEM((2, sc, 2, _CHUNK), jnp.int32),
            pltpu.VMEM((nbuf, _CHUNK, d), jnp.float32),
            pltpu.VMEM_SHARED((acc_rows, d), jnp.float32),
            [pltpu.SemaphoreType.DMA] * nbuf,
            [pltpu.SemaphoreType.DMA] * 2,
        ],
        compiler_params=pltpu.CompilerParams(use_tc_tiling_on_sc=False),
    )
    def k(*refs):
        tables = refs[:n_pass]
        idx_h, z_h, out_h = refs[n_pass:n_pass + 3]
        idx_v, rows_v, acc, sems, isems = refs[n_pass + 3:]
        c = lax.axis_index("c")
        s = lax.axis_index("s")
        w = s * _NC + c
        row0 = s * rows_per_tile

        def iload(sup, sb):
            pltpu.async_copy(idx_h.at[w, pl.ds(sup * sc, sc)],
                             idx_v.at[sb], isems[sb])

        def iwait(sup, sb):
            pltpu.make_async_copy(idx_h.at[w, pl.ds(sup * sc, sc)],
                                  idx_v.at[sb], isems[sb]).wait()

        def gather(p, sb, jj, b):
            pltpu.async_copy(tables[p].at[idx_v.at[sb, jj, 0]],
                             rows_v.at[b], sems[b])

        def gwait(p, sb, jj, b):
            pltpu.make_async_copy(tables[p].at[idx_v.at[sb, jj, 0]],
                                  rows_v.at[b], sems[b]).wait()

        def scatter(sb, jj, b):
            pltpu.sync_copy(rows_v.at[b], acc.at[idx_v.at[sb, jj, 1]],
                            add=True)

        n_blk = rows_per_tile // _CHUNK
        n_blk15 = (acc_rows - 15 * rows_per_tile) // _CHUNK

        for p in range(n_pass):
            pltpu.sync_copy(z_h, rows_v.at[0])
            for r in range(n_blk):
                @pl.when(row0 + (r + 1) * _CHUNK <= acc_rows)
                def _():
                    pltpu.async_copy(
                        rows_v.at[0],
                        acc.at[pl.ds(row0 + r * _CHUNK, _CHUNK)], sems[1])

            def zdrain(i, _):
                pltpu.make_async_copy(rows_v.at[0],
                                      acc.at[pl.ds(0, _CHUNK)], sems[1]).wait()
                return 0

            lax.fori_loop(0, jnp.where(s == _NS - 1, n_blk15, n_blk),
                          zdrain, 0)
            iload(0, 0)
            plsc.subcore_barrier()

            for sup in range(n_super):
                sb = sup % 2
                if sup + 1 < n_super:
                    iload(sup + 1, 1 - sb)
                iwait(sup, sb)
                for b in range(nbuf):
                    gather(p, sb, b, b)

                def inner(i, _):
                    for b in range(nbuf):
                        jj = nbuf * i + b
                        gwait(p, sb, jj, b)
                        scatter(sb, jj, b)
                        gather(p, sb, jj + nbuf, b)
                    return 0

                lax.fori_loop(0, sc // nbuf - 1, inner, 0)
                for b in range(nbuf):
                    jj = sc - nbuf + b
                    gwait(p, sb, jj, b)
                    scatter(sb, jj, b)
            plsc.subcore_barrier()

            for r in range(rows_per_tile // _CHUNK):
                rr = r % 2
                base = row0 + r * _CHUNK

                @pl.when(base + _CHUNK <= n_nodes)
                def _():
                    sl = pl.ds(base, _CHUNK)
                    if r >= 2:
                        pltpu.make_async_copy(
                            rows_v.at[rr], out_h.at[p, c, sl], sems[rr]).wait()
                    pltpu.sync_copy(acc.at[sl], rows_v.at[rr])
                    pltpu.async_copy(rows_v.at[rr], out_h.at[p, c, sl],
                                     sems[rr])

            @pl.when(s == _NS - 1)
            def _():
                tl = pl.ds(n_nodes - 80, 80)
                pltpu.sync_copy(acc.at[tl], rows_v.at[2, pl.ds(0, 80)])
                pltpu.async_copy(rows_v.at[2, pl.ds(0, 80)],
                                 out_h.at[p, c, tl], sems[2])

            for rr in range(2):
                pltpu.make_async_copy(
                    rows_v.at[rr],
                    out_h.at[p, c, pl.ds(row0, _CHUNK)], sems[rr]).wait()

            @pl.when(s == _NS - 1)
            def _():
                pltpu.make_async_copy(
                    rows_v.at[2, pl.ds(0, 80)],
                    out_h.at[p, c, pl.ds(n_nodes - 80, 80)], sems[2]).wait()

    return k


def _dense1(x_ref, w_ref, b_ref, oa_ref, ob_ref, oc_ref, obc_ref):
    y = jnp.dot(x_ref[...], w_ref[...], preferred_element_type=jnp.float32)
    y = y + b_ref[...]
    oa_ref[...] = y[:, 0:32]
    ob_ref[...] = y[:, 32:64]
    oc_ref[...] = y[:, 64:96]
    obc_ref[...] = y[:, 96:240]


def _mid(s_ref, bc_ref, w_ref, b_ref, t2_ref, p2_ref):
    s = s_ref[...]
    s0 = s[0, 0] + s[0, 1]
    s1 = s[1, 0] + s[1, 1]
    s2 = s[2, 0] + s[2, 1]
    deg = s0[:, 24:25]
    agg = jnp.concatenate([s0[:, 0:24], s1[:, 0:24], s2[:, 0:24]], axis=1)
    bc = bc_ref[...]
    h = bc[:, 72:144] - deg * bc[:, 0:72] + agg
    h = jnp.where(h > 0, h, jnp.exp(h) - 1.0)
    y = jnp.dot(h, w_ref[...], preferred_element_type=jnp.float32)
    y = y + b_ref[...]
    t2_ref[...] = y[:, 0:32]
    p2_ref[...] = y[:, 51:70] - deg * y[:, 32:51]


def _out(s_ref, p_ref, o_ref):
    s = s_ref[...]
    logits = p_ref[...] + (s[0, 0] + s[0, 1])[:, 0:19]
    m = jnp.max(logits, axis=1, keepdims=True)
    z = logits - m
    o_ref[...] = z - jnp.log(jnp.sum(jnp.exp(z), axis=1, keepdims=True))


def kernel(x, edge_index, w1_1, b1_1, w2_1, w3_1, b3_1,
           w1_2, b1_2, w2_2, w3_2, b3_2):
    n, f = x.shape
    e = edge_index.shape[1]
    nw = _NC * _NS
    jobs = -(-e // (nw * _CHUNK))
    e_pad = nw * jobs * _CHUNK
    grid = (n // _ROWS,)

    pad_src = jnp.zeros((e_pad - e,), jnp.int32)
    pad_dst = jnp.full((e_pad - e,), n, jnp.int32)
    src = jnp.concatenate([edge_index[0].astype(jnp.int32), pad_src])
    dst = jnp.concatenate([edge_index[1].astype(jnp.int32), pad_dst])
    idx = jnp.stack([src.reshape(nw, jobs, _CHUNK),
                     dst.reshape(nw, jobs, _CHUNK)], axis=2)
    zf = lambda *sh: jnp.zeros(sh, jnp.float32)

    w_a = jnp.concatenate(
        [w1_1[:, 0:24], zf(f, 8), w1_1[:, 24:48], zf(f, 8),
         w1_1[:, 48:72], zf(f, 8), w2_1, w3_1], axis=1)
    one = jnp.ones((1,), jnp.float32)
    b_a = jnp.concatenate(
        [b1_1[0:24], one, zf(7), b1_1[24:48], zf(8),
         b1_1[48:72], zf(8), zf(72), b3_1])[None]

    w_b = jnp.concatenate([w1_2, zf(72, 13), w2_2, w3_2], axis=1)
    b_b = jnp.concatenate([b1_2, zf(13 + 19), b3_2])[None]

    ta, tb, tc, bc = pl.pallas_call(
        _dense1,
        grid=grid,
        in_specs=[
            pl.BlockSpec((_ROWS, f), lambda i: (i, 0)),
            pl.BlockSpec((f, 240), lambda i: (0, 0)),
            pl.BlockSpec((1, 240), lambda i: (0, 0)),
        ],
        out_specs=[
            pl.BlockSpec((_ROWS, 32), lambda i: (i, 0)),
            pl.BlockSpec((_ROWS, 32), lambda i: (i, 0)),
            pl.BlockSpec((_ROWS, 32), lambda i: (i, 0)),
            pl.BlockSpec((_ROWS, 144), lambda i: (i, 0)),
        ],
        out_shape=[
            jax.ShapeDtypeStruct((n, 32), jnp.float32),
            jax.ShapeDtypeStruct((n, 32), jnp.float32),
            jax.ShapeDtypeStruct((n, 32), jnp.float32),
            jax.ShapeDtypeStruct((n, 144), jnp.float32),
        ],
    )(x, w_a, b_a)

    s1 = _sc_scatter(n, 32, 3, jobs)(ta, tb, tc, idx, zf(_CHUNK, 32))

    t2, p2 = pl.pallas_call(
        _mid,
        grid=grid,
        in_specs=[
            pl.BlockSpec((3, 2, _ROWS, 32), lambda i: (0, 0, i, 0)),
            pl.BlockSpec((_ROWS, 144), lambda i: (i, 0)),
            pl.BlockSpec((72, 70), lambda i: (0, 0)),
            pl.BlockSpec((1, 70), lambda i: (0, 0)),
        ],
        out_specs=[
            pl.BlockSpec((_ROWS, 32), lambda i: (i, 0)),
            pl.BlockSpec((_ROWS, 19), lambda i: (i, 0)),
        ],
        out_shape=[
            jax.ShapeDtypeStruct((n, 32), jnp.float32),
            jax.ShapeDtypeStruct((n, 19), jnp.float32),
        ],
    )(s1, bc, w_b, b_b)

    s2 = _sc_scatter(n, 32, 1, jobs)(t2, idx, zf(_CHUNK, 32))

    return pl.pallas_call(
        _out,
        grid=grid,
        in_specs=[
            pl.BlockSpec((1, 2, _ROWS, 32), lambda i: (0, 0, i, 0)),
            pl.BlockSpec((_ROWS, 19), lambda i: (i, 0)),
        ],
        out_specs=pl.BlockSpec((_ROWS, 19), lambda i: (i, 0)),
        out_shape=jax.ShapeDtypeStruct((n, 19), jnp.float32),
    )(s2, p2)
